# Initial kernel scaffold; baseline (speedup 1.0000x reference)
import jax
import jax.numpy as jnp
from jax.experimental import pallas as pl


def kernel(x, positions, pe):
    raise NotImplementedError("write your pallas kernel here")


# speedup vs baseline: 39.9042x; 39.9042x over previous
import functools

import jax
import jax.numpy as jnp
from jax import lax
from jax.experimental import pallas as pl
from jax.experimental.pallas import tpu as pltpu
from jax.experimental.pallas import tpu_sc as plsc

_BATCH = 4096
_SEQ = 200
_D = 64
_TABLE = 201
_KPAD = 208

_INFO = plsc.get_sparse_core_info()
_NC = _INFO.num_cores
_NS = _INFO.num_subcores
_LANES = _INFO.num_lanes
_NW = _NC * _NS
_ROWS = _BATCH // _NW
_GROUPS = _ROWS // _LANES

_mesh = plsc.VectorSubcoreMesh(core_axis_name="c", subcore_axis_name="s")


@functools.partial(
    pl.kernel,
    out_type=jax.ShapeDtypeStruct((_BATCH * _KPAD,), jnp.float32),
    mesh=_mesh,
    compiler_params=pltpu.CompilerParams(needs_layout_passes=False),
    scratch_types=[
        pltpu.VMEM((_ROWS * _SEQ,), jnp.int32),
        pltpu.VMEM((_LANES * _KPAD,), jnp.float32),
    ],
)
def _histogram(pos_hbm, counts_hbm, pos_v, cnt_v):
    wid = lax.axis_index("s") * _NC + lax.axis_index("c")

    pltpu.sync_copy(pos_hbm.at[pl.ds(wid * (_ROWS * _SEQ), _ROWS * _SEQ)], pos_v)

    lane = lax.iota(jnp.int32, _LANES)
    scat_base = lane * _KPAD
    ones = jnp.ones((_LANES,), jnp.float32)
    zeros = jnp.zeros((_LANES,), jnp.float32)

    def group_body(g, _):
        def zero_body(i, _):
            cnt_v[pl.ds(i * _LANES, _LANES)] = zeros
            return 0
        lax.fori_loop(0, (_LANES * _KPAD) // _LANES, zero_body, 0)

        row_base = (g * _LANES + lane) * _SEQ

        def j_body(j, _):
            p = plsc.load_gather(pos_v, [row_base + j])
            ok = plsc.bitcast(p, jnp.uint32) < jnp.uint32(_TABLE)
            plsc.addupdate_scatter(cnt_v, [scat_base + p], ones, mask=ok)
            return 0
        lax.fori_loop(0, _SEQ, j_body, 0)

        out_off = (wid * _ROWS + g * _LANES) * _KPAD
        pltpu.sync_copy(cnt_v, counts_hbm.at[pl.ds(out_off, _LANES * _KPAD)])
        return 0

    lax.fori_loop(0, _GROUPS, group_body, 0)


def _tc_body(x_ref, c_ref, pe_ref, o_ref):
    o_ref[...] = x_ref[...] + jnp.dot(
        c_ref[...], pe_ref[...], preferred_element_type=jnp.float32
    )


def kernel(x, positions, pe):
    pos_flat = positions.astype(jnp.int32).reshape(-1)
    counts = _histogram(pos_flat).reshape(_BATCH, _KPAD)
    pe_pad = jnp.concatenate(
        [pe[0], jnp.zeros((_KPAD - _TABLE, _D), pe.dtype)], axis=0
    )

    grid = 8
    rows = _BATCH // grid
    out = pl.pallas_call(
        _tc_body,
        grid=(grid,),
        in_specs=[
            pl.BlockSpec((rows, _D), lambda i: (i, 0)),
            pl.BlockSpec((rows, _KPAD), lambda i: (i, 0)),
            pl.BlockSpec((_KPAD, _D), lambda i: (0, 0)),
        ],
        out_specs=pl.BlockSpec((rows, _D), lambda i: (i, 0)),
        out_shape=jax.ShapeDtypeStruct((_BATCH, _D), jnp.float32),
    )(x, counts, pe_pad)
    return out


# speedup vs baseline: 42.3975x; 1.0625x over previous
import functools

import jax
import jax.numpy as jnp
from jax import lax
from jax.experimental import pallas as pl
from jax.experimental.pallas import tpu as pltpu
from jax.experimental.pallas import tpu_sc as plsc

_BATCH = 4096
_SEQ = 200
_D = 64
_TABLE = 201
_KPAD = 208

_INFO = plsc.get_sparse_core_info()
_NC = _INFO.num_cores
_NS = _INFO.num_subcores
_LANES = _INFO.num_lanes
_NW = _NC * _NS
_ROWS = _BATCH // _NW
_GROUPS = _ROWS // _LANES

_mesh = plsc.VectorSubcoreMesh(core_axis_name="c", subcore_axis_name="s")


@functools.partial(
    pl.kernel,
    out_type=jax.ShapeDtypeStruct((_BATCH, _KPAD), jnp.float32),
    mesh=_mesh,
    compiler_params=pltpu.CompilerParams(needs_layout_passes=False),
    scratch_types=[
        pltpu.VMEM((_ROWS, _SEQ), jnp.int32),
        pltpu.VMEM((_ROWS, _KPAD), jnp.float32),
    ],
)
def _histogram(pos_hbm, counts_hbm, pos_v, cnt_v):
    wid = lax.axis_index("s") * _NC + lax.axis_index("c")
    base = wid * _ROWS

    pltpu.sync_copy(pos_hbm.at[pl.ds(base, _ROWS), :], pos_v)

    lane = lax.iota(jnp.int32, _LANES)
    ones = jnp.ones((_LANES,), jnp.float32)
    zeros = jnp.zeros((_LANES,), jnp.float32)

    def zero_body(r, _):
        for u in range(_KPAD // _LANES):
            cnt_v[r, pl.ds(u * _LANES, _LANES)] = zeros
        return 0
    lax.fori_loop(0, _ROWS, zero_body, 0)

    row_vecs = [(jnp.int32(g * _LANES) + lane) for g in range(_GROUPS)]

    def j_body(j, _):
        col = jnp.full((_LANES,), j, jnp.int32)
        for g in range(_GROUPS):
            p = plsc.load_gather(pos_v, [row_vecs[g], col])
            ok = plsc.bitcast(p, jnp.uint32) < jnp.uint32(_TABLE)
            plsc.addupdate_scatter(cnt_v, [row_vecs[g], p], ones, mask=ok)
        return 0
    lax.fori_loop(0, _SEQ, j_body, 0)

    pltpu.sync_copy(cnt_v, counts_hbm.at[pl.ds(base, _ROWS), :])


def _tc_body(x_ref, c_ref, pe_ref, o_ref):
    o_ref[...] = x_ref[...] + jnp.dot(
        c_ref[...], pe_ref[...], preferred_element_type=jnp.float32
    )


def kernel(x, positions, pe):
    counts = _histogram(positions.astype(jnp.int32))
    pe_pad = jnp.concatenate(
        [pe[0], jnp.zeros((_KPAD - _TABLE, _D), pe.dtype)], axis=0
    )

    grid = 8
    rows = _BATCH // grid
    out = pl.pallas_call(
        _tc_body,
        grid=(grid,),
        in_specs=[
            pl.BlockSpec((rows, _D), lambda i: (i, 0)),
            pl.BlockSpec((rows, _KPAD), lambda i: (i, 0)),
            pl.BlockSpec((_KPAD, _D), lambda i: (0, 0)),
        ],
        out_specs=pl.BlockSpec((rows, _D), lambda i: (i, 0)),
        out_shape=jax.ShapeDtypeStruct((_BATCH, _D), jnp.float32),
    )(x, counts, pe_pad)
    return out


# speedup vs baseline: 53.3531x; 1.2584x over previous
import functools

import jax
import jax.numpy as jnp
from jax import lax
from jax.experimental import pallas as pl
from jax.experimental.pallas import tpu as pltpu
from jax.experimental.pallas import tpu_sc as plsc

_BATCH = 4096
_SEQ = 200
_D = 64
_TABLE = 201
_KPAD = 208

_INFO = plsc.get_sparse_core_info()
_NC = _INFO.num_cores
_NS = _INFO.num_subcores
_LANES = _INFO.num_lanes
_NW = _NC * _NS
_ROWS = _BATCH // _NW
_GROUPS = _ROWS // _LANES

_mesh = plsc.VectorSubcoreMesh(core_axis_name="c", subcore_axis_name="s")


@functools.partial(
    pl.kernel,
    out_type=jax.ShapeDtypeStruct((_BATCH, _KPAD), jnp.float32),
    mesh=_mesh,
    compiler_params=pltpu.CompilerParams(needs_layout_passes=False),
    scratch_types=[
        pltpu.VMEM((_ROWS, _SEQ), jnp.int32),
        pltpu.VMEM((_ROWS, _KPAD), jnp.float32),
    ],
)
def _histogram(pos_hbm, counts_hbm, pos_v, cnt_v):
    wid = lax.axis_index("s") * _NC + lax.axis_index("c")
    base = wid * _ROWS

    pltpu.sync_copy(pos_hbm.at[pl.ds(base, _ROWS), :], pos_v)

    lane = lax.iota(jnp.int32, _LANES)
    ones = jnp.ones((_LANES,), jnp.float32)
    zeros = jnp.zeros((_LANES,), jnp.float32)
    nfull = _SEQ // _LANES
    tail = _SEQ - nfull * _LANES
    tail_lanes = lane < tail

    def row_body(r, _):
        for u in range(_KPAD // _LANES):
            cnt_v[r, pl.ds(u * _LANES, _LANES)] = zeros
        row = jnp.full((_LANES,), r, jnp.int32)
        for c in range(nfull):
            p = pos_v[r, pl.ds(c * _LANES, _LANES)]
            ok = plsc.bitcast(p, jnp.uint32) < jnp.uint32(_TABLE)
            plsc.addupdate_scatter(cnt_v, [row, p], ones, mask=ok)
        toff = jnp.minimum(nfull * _LANES + lane, _SEQ - 1)
        p = plsc.load_gather(pos_v, [row, toff])
        ok = (plsc.bitcast(p, jnp.uint32) < jnp.uint32(_TABLE)) & tail_lanes
        plsc.addupdate_scatter(cnt_v, [row, p], ones, mask=ok)
        return 0
    lax.fori_loop(0, _ROWS, row_body, 0)

    pltpu.sync_copy(cnt_v, counts_hbm.at[pl.ds(base, _ROWS), :])


def _tc_body(x_ref, c_ref, pe_ref, o_ref):
    o_ref[...] = x_ref[...] + jnp.dot(
        c_ref[...], pe_ref[...], preferred_element_type=jnp.float32
    )


def kernel(x, positions, pe):
    counts = _histogram(positions.astype(jnp.int32))
    pe_pad = jnp.concatenate(
        [pe[0], jnp.zeros((_KPAD - _TABLE, _D), pe.dtype)], axis=0
    )

    grid = 8
    rows = _BATCH // grid
    out = pl.pallas_call(
        _tc_body,
        grid=(grid,),
        in_specs=[
            pl.BlockSpec((rows, _D), lambda i: (i, 0)),
            pl.BlockSpec((rows, _KPAD), lambda i: (i, 0)),
            pl.BlockSpec((_KPAD, _D), lambda i: (0, 0)),
        ],
        out_specs=pl.BlockSpec((rows, _D), lambda i: (i, 0)),
        out_shape=jax.ShapeDtypeStruct((_BATCH, _D), jnp.float32),
    )(x, counts, pe_pad)
    return out


# speedup vs baseline: 58.9729x; 1.1053x over previous
import functools

import jax
import jax.numpy as jnp
from jax import lax
from jax.experimental import pallas as pl
from jax.experimental.pallas import tpu as pltpu
from jax.experimental.pallas import tpu_sc as plsc

_BATCH = 4096
_SEQ = 200
_D = 64
_TABLE = 201
_KPAD = 208

_INFO = plsc.get_sparse_core_info()
_NC = _INFO.num_cores
_NS = _INFO.num_subcores
_LANES = _INFO.num_lanes
_NW = _NC * _NS
_ROWS = _BATCH // _NW
_GROUPS = _ROWS // _LANES

_mesh = plsc.VectorSubcoreMesh(core_axis_name="c", subcore_axis_name="s")


@functools.partial(
    pl.kernel,
    out_type=jax.ShapeDtypeStruct((_BATCH, _KPAD), jnp.float32),
    mesh=_mesh,
    compiler_params=pltpu.CompilerParams(needs_layout_passes=False),
    scratch_types=[
        pltpu.VMEM((_ROWS, _SEQ), jnp.int32),
        pltpu.VMEM((_ROWS, _KPAD), jnp.float32),
    ],
)
def _histogram(pos_hbm, counts_hbm, pos_v, cnt_v):
    wid = lax.axis_index("s") * _NC + lax.axis_index("c")
    base = wid * _ROWS

    pltpu.sync_copy(pos_hbm.at[pl.ds(base, _ROWS), :], pos_v)

    lane = lax.iota(jnp.int32, _LANES)
    ones = jnp.ones((_LANES,), jnp.float32)
    zeros = jnp.zeros((_LANES,), jnp.float32)
    nfull = _SEQ // _LANES
    tail = _SEQ - nfull * _LANES
    tail_lanes = lane < tail

    _UNROLL = 2
    toff = jnp.minimum(nfull * _LANES + lane, _SEQ - 1)

    def row_body(i, _):
        rows = [i * _UNROLL + k for k in range(_UNROLL)]
        for r in rows:
            for u in range(_KPAD // _LANES):
                cnt_v[r, pl.ds(u * _LANES, _LANES)] = zeros
        rvecs = [jnp.full((_LANES,), r, jnp.int32) for r in rows]
        for c in range(nfull):
            ps = [pos_v[r, pl.ds(c * _LANES, _LANES)] for r in rows]
            for rv, p in zip(rvecs, ps):
                ok = plsc.bitcast(p, jnp.uint32) < jnp.uint32(_TABLE)
                plsc.addupdate_scatter(cnt_v, [rv, p], ones, mask=ok)
        for rv in rvecs:
            p = plsc.load_gather(pos_v, [rv, toff])
            ok = (plsc.bitcast(p, jnp.uint32) < jnp.uint32(_TABLE)) & tail_lanes
            plsc.addupdate_scatter(cnt_v, [rv, p], ones, mask=ok)
        return 0
    lax.fori_loop(0, _ROWS // _UNROLL, row_body, 0)

    pltpu.sync_copy(cnt_v, counts_hbm.at[pl.ds(base, _ROWS), :])


def _tc_body(x_ref, c_ref, pe_ref, o_ref):
    o_ref[...] = x_ref[...] + jnp.dot(
        c_ref[...], pe_ref[...], preferred_element_type=jnp.float32
    )


def kernel(x, positions, pe):
    counts = _histogram(positions.astype(jnp.int32))
    pe_pad = jnp.concatenate(
        [pe[0], jnp.zeros((_KPAD - _TABLE, _D), pe.dtype)], axis=0
    )

    grid = 8
    rows = _BATCH // grid
    out = pl.pallas_call(
        _tc_body,
        grid=(grid,),
        in_specs=[
            pl.BlockSpec((rows, _D), lambda i: (i, 0)),
            pl.BlockSpec((rows, _KPAD), lambda i: (i, 0)),
            pl.BlockSpec((_KPAD, _D), lambda i: (0, 0)),
        ],
        out_specs=pl.BlockSpec((rows, _D), lambda i: (i, 0)),
        out_shape=jax.ShapeDtypeStruct((_BATCH, _D), jnp.float32),
    )(x, counts, pe_pad)
    return out


# speedup vs baseline: 61.9752x; 1.0509x over previous
import functools

import jax
import jax.numpy as jnp
from jax import lax
from jax.experimental import pallas as pl
from jax.experimental.pallas import tpu as pltpu
from jax.experimental.pallas import tpu_sc as plsc

_BATCH = 4096
_SEQ = 200
_D = 64
_TABLE = 201
_KPAD = 208

_INFO = plsc.get_sparse_core_info()
_NC = _INFO.num_cores
_NS = _INFO.num_subcores
_LANES = _INFO.num_lanes
_NW = _NC * _NS
_ROWS = _BATCH // _NW
_GROUPS = _ROWS // _LANES

_mesh = plsc.VectorSubcoreMesh(core_axis_name="c", subcore_axis_name="s")


@functools.partial(
    pl.kernel,
    out_type=jax.ShapeDtypeStruct((_BATCH, _KPAD), jnp.float32),
    mesh=_mesh,
    compiler_params=pltpu.CompilerParams(needs_layout_passes=False),
    scratch_types=[
        pltpu.VMEM((_ROWS, _SEQ), jnp.int32),
        pltpu.VMEM((_ROWS, _KPAD), jnp.float32),
    ],
)
def _histogram(pos_hbm, counts_hbm, pos_v, cnt_v):
    wid = lax.axis_index("s") * _NC + lax.axis_index("c")
    base = wid * _ROWS

    pltpu.sync_copy(pos_hbm.at[pl.ds(base, _ROWS), :], pos_v)

    lane = lax.iota(jnp.int32, _LANES)
    ones = jnp.ones((_LANES,), jnp.float32)
    zeros = jnp.zeros((_LANES,), jnp.float32)
    nfull = _SEQ // _LANES
    tail = _SEQ - nfull * _LANES
    tail_lanes = lane < tail

    _UNROLL = 4
    toff = jnp.minimum(nfull * _LANES + lane, _SEQ - 1)

    def row_body(i, _):
        rows = [i * _UNROLL + k for k in range(_UNROLL)]
        for r in rows:
            for u in range(_KPAD // _LANES):
                cnt_v[r, pl.ds(u * _LANES, _LANES)] = zeros
        rvecs = [jnp.full((_LANES,), r, jnp.int32) for r in rows]
        for c in range(nfull):
            ps = [pos_v[r, pl.ds(c * _LANES, _LANES)] for r in rows]
            for rv, p in zip(rvecs, ps):
                ok = plsc.bitcast(p, jnp.uint32) < jnp.uint32(_TABLE)
                plsc.addupdate_scatter(cnt_v, [rv, p], ones, mask=ok)
        for rv in rvecs:
            p = plsc.load_gather(pos_v, [rv, toff])
            ok = (plsc.bitcast(p, jnp.uint32) < jnp.uint32(_TABLE)) & tail_lanes
            plsc.addupdate_scatter(cnt_v, [rv, p], ones, mask=ok)
        return 0
    lax.fori_loop(0, _ROWS // _UNROLL, row_body, 0)

    pltpu.sync_copy(cnt_v, counts_hbm.at[pl.ds(base, _ROWS), :])


def _tc_body(x_ref, c_ref, pe_ref, o_ref):
    o_ref[...] = x_ref[...] + jnp.dot(
        c_ref[...], pe_ref[...], preferred_element_type=jnp.float32
    )


def kernel(x, positions, pe):
    counts = _histogram(positions.astype(jnp.int32))
    pe_pad = jnp.concatenate(
        [pe[0], jnp.zeros((_KPAD - _TABLE, _D), pe.dtype)], axis=0
    )

    grid = 8
    rows = _BATCH // grid
    out = pl.pallas_call(
        _tc_body,
        grid=(grid,),
        in_specs=[
            pl.BlockSpec((rows, _D), lambda i: (i, 0)),
            pl.BlockSpec((rows, _KPAD), lambda i: (i, 0)),
            pl.BlockSpec((_KPAD, _D), lambda i: (0, 0)),
        ],
        out_specs=pl.BlockSpec((rows, _D), lambda i: (i, 0)),
        out_shape=jax.ShapeDtypeStruct((_BATCH, _D), jnp.float32),
    )(x, counts, pe_pad)
    return out


# speedup vs baseline: 64.3065x; 1.0376x over previous
import functools

import jax
import jax.numpy as jnp
from jax import lax
from jax.experimental import pallas as pl
from jax.experimental.pallas import tpu as pltpu
from jax.experimental.pallas import tpu_sc as plsc

_BATCH = 4096
_SEQ = 200
_D = 64
_TABLE = 201
_KPAD = 208

_INFO = plsc.get_sparse_core_info()
_NC = _INFO.num_cores
_NS = _INFO.num_subcores
_LANES = _INFO.num_lanes
_NW = _NC * _NS
_ROWS = _BATCH // _NW
_GROUPS = _ROWS // _LANES

_mesh = plsc.VectorSubcoreMesh(core_axis_name="c", subcore_axis_name="s")


@functools.partial(
    pl.kernel,
    out_type=jax.ShapeDtypeStruct((_BATCH, _KPAD), jnp.float32),
    mesh=_mesh,
    compiler_params=pltpu.CompilerParams(needs_layout_passes=False),
    scratch_types=[
        pltpu.VMEM((_ROWS, _SEQ), jnp.int32),
        pltpu.VMEM((_ROWS, _KPAD), jnp.float32),
    ],
)
def _histogram(pos_hbm, counts_hbm, pos_v, cnt_v):
    wid = lax.axis_index("s") * _NC + lax.axis_index("c")
    base = wid * _ROWS

    pltpu.sync_copy(pos_hbm.at[pl.ds(base, _ROWS), :], pos_v)

    lane = lax.iota(jnp.int32, _LANES)
    ones = jnp.ones((_LANES,), jnp.float32)
    zeros = jnp.zeros((_LANES,), jnp.float32)
    nfull = _SEQ // _LANES
    tail = _SEQ - nfull * _LANES
    tail_lanes = lane < tail

    _UNROLL = 8
    toff = jnp.minimum(nfull * _LANES + lane, _SEQ - 1)

    def row_body(i, _):
        rows = [i * _UNROLL + k for k in range(_UNROLL)]
        for r in rows:
            for u in range(_KPAD // _LANES):
                cnt_v[r, pl.ds(u * _LANES, _LANES)] = zeros
        rvecs = [jnp.full((_LANES,), r, jnp.int32) for r in rows]
        for c in range(nfull):
            ps = [pos_v[r, pl.ds(c * _LANES, _LANES)] for r in rows]
            for rv, p in zip(rvecs, ps):
                ok = plsc.bitcast(p, jnp.uint32) < jnp.uint32(_TABLE)
                plsc.addupdate_scatter(cnt_v, [rv, p], ones, mask=ok)
        for rv in rvecs:
            p = plsc.load_gather(pos_v, [rv, toff])
            ok = (plsc.bitcast(p, jnp.uint32) < jnp.uint32(_TABLE)) & tail_lanes
            plsc.addupdate_scatter(cnt_v, [rv, p], ones, mask=ok)
        return 0
    lax.fori_loop(0, _ROWS // _UNROLL, row_body, 0)

    pltpu.sync_copy(cnt_v, counts_hbm.at[pl.ds(base, _ROWS), :])


def _tc_body(x_ref, c_ref, pe_ref, o_ref):
    o_ref[...] = x_ref[...] + jnp.dot(
        c_ref[...], pe_ref[...], preferred_element_type=jnp.float32
    )


def kernel(x, positions, pe):
    counts = _histogram(positions.astype(jnp.int32))
    pe_pad = jnp.concatenate(
        [pe[0], jnp.zeros((_KPAD - _TABLE, _D), pe.dtype)], axis=0
    )

    grid = 8
    rows = _BATCH // grid
    out = pl.pallas_call(
        _tc_body,
        grid=(grid,),
        in_specs=[
            pl.BlockSpec((rows, _D), lambda i: (i, 0)),
            pl.BlockSpec((rows, _KPAD), lambda i: (i, 0)),
            pl.BlockSpec((_KPAD, _D), lambda i: (0, 0)),
        ],
        out_specs=pl.BlockSpec((rows, _D), lambda i: (i, 0)),
        out_shape=jax.ShapeDtypeStruct((_BATCH, _D), jnp.float32),
    )(x, counts, pe_pad)
    return out
